# Initial kernel scaffold; baseline (speedup 1.0000x reference)
import jax
import jax.numpy as jnp
from jax.experimental import pallas as pl


def kernel(sequence_output, pooled_output, token_type_ids, word_mask, gap_ids, W_gap, b_gap, W_cls, b_cls):
    raise NotImplementedError("write your pallas kernel here")


# speedup vs baseline: 7.2880x; 7.2880x over previous
import jax
import jax.numpy as jnp
from jax import lax
from jax.experimental import pallas as pl
from jax.experimental.pallas import tpu as pltpu

WIN = 15
WLEN = 2 * WIN + 1
WPAD = 40
BS = 512


def _body(gap_ref, bgap_ref, bcls_ref,
          x_ref, bm_ref, pooled_ref, wg_ref, wc_ref,
          out_ref,
          wmax_ref, wsum_ref, gapv_ref, tmax_ref, tsum_ref, cnt_ref, tcnt_ref):
    b = pl.program_id(0)
    s = pl.program_id(1)
    ns = pl.num_programs(1)
    H = x_ref.shape[2]
    G = wmax_ref.shape[0]
    base = s * BS

    @pl.when(s == 0)
    def _init():
        wmax_ref[...] = jnp.zeros_like(wmax_ref)
        wsum_ref[...] = jnp.zeros_like(wsum_ref)
        gapv_ref[...] = jnp.zeros_like(gapv_ref)
        tsum_ref[...] = jnp.zeros_like(tsum_ref)
        tmax_ref[...] = jnp.full_like(tmax_ref, -jnp.inf)
        cnt_ref[...] = jnp.zeros_like(cnt_ref)
        tcnt_ref[...] = jnp.zeros_like(tcnt_ref)

    x = x_ref[0]
    bm = bm_ref[0, :, :]
    xb = x * bm
    tmax_ref[...] = jnp.maximum(tmax_ref[...], jnp.max(xb, axis=0, keepdims=True))
    tsum_ref[...] = tsum_ref[...] + jnp.sum(xb, axis=0, keepdims=True)
    tcnt_ref[...] = tcnt_ref[...] + jnp.sum(bm)

    for g in range(G):
        gid = gap_ref[b, g]
        lo = gid - WIN
        hi = gid + WIN

        @pl.when(jnp.logical_and(hi >= base, lo <= base + BS - 1))
        def _acc(g=g, gid=gid, lo=lo, hi=hi):
            d = jnp.clip(lo - base, 0, BS - WLEN)
            d = pl.multiple_of(jnp.minimum((d // 8) * 8, BS - WPAD), 8)
            sl = x_ref[0, pl.ds(d, WPAD), :]
            bmr = bm_ref[0, pl.ds(d, WPAD), :]
            pos = base + d + lax.broadcasted_iota(jnp.int32, (WPAD, 1), 0)
            inwin = jnp.logical_and(pos >= lo, pos <= hi).astype(jnp.float32)
            rowm = inwin * bmr
            m = sl * rowm
            wmax_ref[g:g + 1, :] = jnp.maximum(
                wmax_ref[g:g + 1, :], jnp.max(m, axis=0, keepdims=True))
            wsum_ref[g:g + 1, :] = wsum_ref[g:g + 1, :] + jnp.sum(m, axis=0, keepdims=True)
            cnt_ref[g:g + 1, :] = cnt_ref[g:g + 1, :] + jnp.sum(rowm)
            gm = (pos == gid).astype(jnp.float32)
            gapv_ref[g:g + 1, :] = gapv_ref[g:g + 1, :] + jnp.sum(sl * gm, axis=0, keepdims=True)

    @pl.when(s == ns - 1)
    def _final():
        wg1 = wg_ref[0:1, 0:H]
        wg2 = wg_ref[0:1, H:2 * H]
        wg3 = wg_ref[0:1, 2 * H:3 * H]
        wc1 = wc_ref[0:1, 0:H]
        wc2 = wc_ref[0:1, H:2 * H]
        wc3 = wc_ref[0:1, 2 * H:3 * H]
        counts = cnt_ref[:, 0:1]
        wavg = wsum_ref[...] / counts
        gap_scores = (jnp.sum(gapv_ref[...] * wg1, axis=1, keepdims=True)
                      + jnp.sum(wmax_ref[...] * wg2, axis=1, keepdims=True)
                      + jnp.sum(wavg * wg3, axis=1, keepdims=True)
                      + bgap_ref[0])
        tavg = tsum_ref[...] / tcnt_ref[0:1, 0:1]
        pooled = pooled_ref[0]
        cls_score = (jnp.sum(pooled * wc1, axis=1, keepdims=True)
                     + jnp.sum(tmax_ref[...] * wc2, axis=1, keepdims=True)
                     + jnp.sum(tavg * wc3, axis=1, keepdims=True)
                     + bcls_ref[0])
        out_ref[0] = jnp.concatenate([cls_score, gap_scores], axis=0)


def kernel(sequence_output, pooled_output, token_type_ids, word_mask, gap_ids,
           W_gap, b_gap, W_cls, b_cls):
    B, S, H = sequence_output.shape
    G = gap_ids.shape[1]
    ns = S // BS
    bm = ((token_type_ids == 0).astype(jnp.int32) * word_mask
          ).astype(jnp.float32)[..., None]
    pooled3 = pooled_output[:, None, :]
    out = pl.pallas_call(
        _body,
        grid=(B, ns),
        in_specs=[
            pl.BlockSpec(memory_space=pltpu.SMEM),
            pl.BlockSpec(memory_space=pltpu.SMEM),
            pl.BlockSpec(memory_space=pltpu.SMEM),
            pl.BlockSpec((1, BS, H), lambda b, s: (b, s, 0)),
            pl.BlockSpec((1, BS, 1), lambda b, s: (b, s, 0)),
            pl.BlockSpec((1, 1, H), lambda b, s: (b, 0, 0)),
            pl.BlockSpec((1, 3 * H), lambda b, s: (0, 0)),
            pl.BlockSpec((1, 3 * H), lambda b, s: (0, 0)),
        ],
        out_specs=pl.BlockSpec((1, 1 + G, 1), lambda b, s: (b, 0, 0)),
        out_shape=jax.ShapeDtypeStruct((B, 1 + G, 1), jnp.float32),
        scratch_shapes=[
            pltpu.VMEM((G, H), jnp.float32),
            pltpu.VMEM((G, H), jnp.float32),
            pltpu.VMEM((G, H), jnp.float32),
            pltpu.VMEM((1, H), jnp.float32),
            pltpu.VMEM((1, H), jnp.float32),
            pltpu.VMEM((G, 128), jnp.float32),
            pltpu.VMEM((1, 128), jnp.float32),
        ],
    )(gap_ids, b_gap, b_cls, sequence_output, bm, pooled3, W_gap, W_cls)
    return out[:, :, 0]


# speedup vs baseline: 9.5180x; 1.3060x over previous
import jax
import jax.numpy as jnp
from jax import lax
from jax.experimental import pallas as pl
from jax.experimental.pallas import tpu as pltpu

WIN = 15
WLEN = 2 * WIN + 1
WPAD = 40


def _body(gap_ref, bgap_ref, bcls_ref,
          x_ref, bm_ref, pooled_ref, wg_ref, wc_ref,
          out_ref):
    b = pl.program_id(0)
    S = x_ref.shape[1]
    H = x_ref.shape[2]
    G = gap_ref.shape[1]

    x = x_ref[0]
    bm = bm_ref[0, :, :]
    xb = x * bm
    tmax = jnp.max(xb, axis=0, keepdims=True)
    tsum = jnp.sum(xb, axis=0, keepdims=True)
    tcnt = jnp.sum(bm)

    wg1 = wg_ref[0:1, 0:H]
    wg2 = wg_ref[0:1, H:2 * H]
    wg3 = wg_ref[0:1, 2 * H:3 * H]
    wc1 = wc_ref[0:1, 0:H]
    wc2 = wc_ref[0:1, H:2 * H]
    wc3 = wc_ref[0:1, 2 * H:3 * H]

    tavg = tsum / tcnt
    pooled = pooled_ref[0]
    cls_score = (jnp.sum(pooled * wc1, axis=1, keepdims=True)
                 + jnp.sum(tmax * wc2, axis=1, keepdims=True)
                 + jnp.sum(tavg * wc3, axis=1, keepdims=True)
                 + bcls_ref[0])

    scores = [cls_score]
    for g in range(G):
        gid = gap_ref[b, g]
        lo = gid - WIN
        hi = gid + WIN
        d = jnp.clip(lo, 0, S - WPAD)
        d = pl.multiple_of(jnp.minimum((d // 8) * 8, S - WPAD), 8)
        sl = x_ref[0, pl.ds(d, WPAD), :]
        bmr = bm_ref[0, pl.ds(d, WPAD), :]
        pos = d + lax.broadcasted_iota(jnp.int32, (WPAD, 1), 0)
        inwin = jnp.logical_and(pos >= lo, pos <= hi).astype(jnp.float32)
        rowm = inwin * bmr
        m = sl * rowm
        wmax = jnp.maximum(jnp.max(m, axis=0, keepdims=True), 0.0)
        wsum = jnp.sum(m, axis=0, keepdims=True)
        cnt = jnp.sum(rowm)
        dg = pl.multiple_of(jnp.minimum((gid // 8) * 8, S - 8), 8)
        rows8 = x_ref[0, pl.ds(dg, 8), :]
        pg = dg + lax.broadcasted_iota(jnp.int32, (8, 1), 0)
        gaprow = jnp.sum(rows8 * (pg == gid).astype(jnp.float32),
                         axis=0, keepdims=True)
        sc = (jnp.sum(gaprow * wg1, axis=1, keepdims=True)
              + jnp.sum(wmax * wg2, axis=1, keepdims=True)
              + jnp.sum((wsum / cnt) * wg3, axis=1, keepdims=True)
              + bgap_ref[0])
        scores.append(sc)

    out_ref[0] = jnp.concatenate(scores, axis=0)


def kernel(sequence_output, pooled_output, token_type_ids, word_mask, gap_ids,
           W_gap, b_gap, W_cls, b_cls):
    B, S, H = sequence_output.shape
    G = gap_ids.shape[1]
    bm = ((token_type_ids == 0).astype(jnp.int32) * word_mask
          ).astype(jnp.float32)[..., None]
    pooled3 = pooled_output[:, None, :]
    out = pl.pallas_call(
        _body,
        grid=(B,),
        in_specs=[
            pl.BlockSpec(memory_space=pltpu.SMEM),
            pl.BlockSpec(memory_space=pltpu.SMEM),
            pl.BlockSpec(memory_space=pltpu.SMEM),
            pl.BlockSpec((1, S, H), lambda b: (b, 0, 0)),
            pl.BlockSpec((1, S, 1), lambda b: (b, 0, 0)),
            pl.BlockSpec((1, 1, H), lambda b: (b, 0, 0)),
            pl.BlockSpec((1, 3 * H), lambda b: (0, 0)),
            pl.BlockSpec((1, 3 * H), lambda b: (0, 0)),
        ],
        out_specs=pl.BlockSpec((1, 1 + G, 1), lambda b: (b, 0, 0)),
        out_shape=jax.ShapeDtypeStruct((B, 1 + G, 1), jnp.float32),
    )(gap_ids, b_gap, b_cls, sequence_output, bm, pooled3, W_gap, W_cls)
    return out[:, :, 0]
